# Initial kernel scaffold; baseline (speedup 1.0000x reference)
import jax
import jax.numpy as jnp
from jax.experimental import pallas as pl


def kernel(query, reference_points, temporal_offsets, input_flatten, input_spatial_shapes, input_level_start_index, Wv, bv, Wso, bso, Waw, baw, Wtso, btso, Wtaw, btaw, Wo, bo):
    raise NotImplementedError("write your pallas kernel here")


# speedup vs baseline: 1.4990x; 1.4990x over previous
import functools

import numpy as np
import jax
import jax.numpy as jnp
from jax import lax
from jax.experimental import pallas as pl
from jax.experimental.pallas import tpu as pltpu
from jax.experimental.pallas import tpu_sc as plsc

_NH, _NL, _NP, _TW, _NTP = 8, 4, 4, 2, 2
_DM = 256
_DH = _DM // _NH
_NPTS = _NL * (_NP + _TW * _NTP)
_NC = 4 * _NPTS


def _build_static():
    lvl = np.zeros(256, np.int32)
    head = np.zeros(256, np.int32)
    dt = np.zeros(256, np.int32)
    perm_lg = np.zeros(256, np.int64)
    perm_x = np.zeros(256, np.int64)
    sel = np.zeros((16, 256), np.float32)
    for c in range(256):
        h, j = divmod(c, 32)
        l, k = divmod(j, 8)
        lvl[c] = l
        head[c] = h
        sel[l, c] = 1.0
        if k < 4:
            p = k
            perm_lg[c] = h * 16 + l * 4 + p
            perm_x[c] = ((h * 4 + l) * 4 + p) * 2
        else:
            w, tp = divmod(k - 4, _NTP)
            dt[c] = -1 if w == 0 else 1
            perm_lg[c] = 128 + h * 16 + w * 8 + l * 2 + tp
            perm_x[c] = 256 + ((((h * 2 + w) * 4 + l) * 2 + tp) * 2)
            sel[4 + w * 4 + l, c] = 1.0
    perm_y = perm_x + 1
    grp = np.arange(256) // 32
    bd = (grp[:, None] == grp[None, :]).astype(np.float32)
    return lvl, head, dt, perm_lg, perm_x, perm_y, sel, bd


_LVL, _HEAD, _DT, _PERM_LG, _PERM_X, _PERM_Y, _SEL, _BD = _build_static()


def _mm_body(x_ref, w_ref, b_ref, o_ref):
    o_ref[...] = jnp.dot(x_ref[...], w_ref[...], precision=lax.Precision.HIGHEST,
                         preferred_element_type=jnp.float32) + b_ref[...]


def _matmul(x, w, b, mb):
    m, kdim = x.shape
    n = w.shape[1]
    return pl.pallas_call(
        _mm_body,
        grid=(m // mb,),
        in_specs=[pl.BlockSpec((mb, kdim), lambda i: (i, 0)),
                  pl.BlockSpec((kdim, n), lambda i: (0, 0)),
                  pl.BlockSpec((1, n), lambda i: (0, 0))],
        out_specs=pl.BlockSpec((mb, n), lambda i: (i, 0)),
        out_shape=jax.ShapeDtypeStruct((m, n), jnp.float32),
    )(x, w, b.reshape(1, n))


def _prep_body(q_ref, wcat_ref, bcat_ref, rtx_ref, rty_ref, sel_ref, bd_ref,
               wl_ref, hl_ref, st_ref, head_ref, dt_ref, tm1_ref, tq_ref,
               idx_ref, wgt_ref):
    p = jnp.dot(q_ref[...], wcat_ref[...], precision=lax.Precision.HIGHEST,
                preferred_element_type=jnp.float32) + bcat_ref[...]
    lg = p[:, :256]
    sx = p[:, 256:512]
    sy = p[:, 512:768]
    m = jnp.max(lg, axis=1, keepdims=True)
    e = jnp.exp(lg - m)
    den = jnp.dot(e, bd_ref[...], precision=lax.Precision.HIGHEST,
                    preferred_element_type=jnp.float32)
    a = e / den
    rx = jnp.dot(rtx_ref[...], sel_ref[...], precision=lax.Precision.HIGHEST,
                 preferred_element_type=jnp.float32)
    ry = jnp.dot(rty_ref[...], sel_ref[...], precision=lax.Precision.HIGHEST,
                 preferred_element_type=jnp.float32)
    wl = wl_ref[...]
    hl = hl_ref[...]
    x = rx * wl + sx - 0.5
    y = ry * hl + sy - 0.5
    x0f = jnp.floor(x)
    y0f = jnp.floor(y)
    x0 = x0f.astype(jnp.int32)
    y0 = y0f.astype(jnp.int32)
    wx1 = x - x0f
    wx0 = 1.0 - wx1
    wy1 = y - y0f
    wy0 = 1.0 - wy1
    wli = wl.astype(jnp.int32)
    hli = hl.astype(jnp.int32)
    t = jnp.clip(tq_ref[...] + dt_ref[...], 0, tm1_ref[...])
    sti = st_ref[...]
    headi = head_ref[...]
    for ci, (dx, dy) in enumerate(((0, 0), (1, 0), (0, 1), (1, 1))):
        xi = x0 + dx
        yi = y0 + dy
        valid = (xi >= 0) & (xi < wli) & (yi >= 0) & (yi < hli)
        xc = jnp.clip(xi, 0, wli - 1)
        yc = jnp.clip(yi, 0, hli - 1)
        flat = jnp.right_shift(
            (sti + (t * hli + yc) * wli + xc) * _NH + headi, 2)
        wx = wx1 if dx else wx0
        wy = wy1 if dy else wy0
        idx_ref[ci] = flat
        wgt_ref[ci] = a * wx * wy * valid.astype(jnp.float32)


def _prep(len_q, q, wcat, bcat, rtx, rty, sel, bd, wlv, hlv, stv, headv, dtv,
          tm1, tq):
    return pl.pallas_call(
        _prep_body,
        out_shape=(jax.ShapeDtypeStruct((4, len_q, 256), jnp.int32),
                   jax.ShapeDtypeStruct((4, len_q, 256), jnp.float32)),
    )(q, wcat, bcat, rtx, rty, sel, bd, wlv, hlv, stv, headv, dtv, tm1, tq)


_NWORK = 32
_CH = 8


_GSZ = 2
_NGRP = _CH // _GSZ


def _make_sc_gather(nrows):
    rpw = nrows // _NWORK
    nchunk = rpw // _CH
    mesh = plsc.VectorSubcoreMesh(core_axis_name="c", subcore_axis_name="s")

    @functools.partial(
        pl.kernel,
        mesh=mesh,
        out_type=jax.ShapeDtypeStruct((nrows, _DH), jnp.float32),
        scratch_types=[
            pltpu.VMEM((_CH, _NC), jnp.int32),
            pltpu.VMEM((_CH, _NC), jnp.float32),
            pltpu.VMEM((_NC, 128), jnp.float32),
            pltpu.VMEM((rpw, _DH), jnp.float32),
            pltpu.SemaphoreType.DMA,
        ],
    )
    def sc_gather(value_hbm, idx_hbm, wgt_hbm, out_hbm,
                  idxb, wgtb, rows, outb, sem0):
        cid = lax.axis_index("c")
        sid = lax.axis_index("s")
        wid = sid * 2 + cid
        base = wid * rpw

        def chunk_body(ch, carry):
            r0 = base + ch * _CH
            pltpu.sync_copy(idx_hbm.at[pl.ds(r0, _CH)], idxb)
            pltpu.sync_copy(wgt_hbm.at[pl.ds(r0, _CH)], wgtb)
            for i in range(_CH):
                pltpu.async_copy(
                    value_hbm.at[idxb.at[i]], rows, sem0).wait()
                off = (i % 4) * _DH

                def inner(gg, accs, _off=off, _i=i):
                    a0, a1 = accs
                    w16 = wgtb[_i, pl.ds(gg * 16, 16)]
                    for lane in range(16):
                        w = w16[lane]
                        kk = gg * 16 + lane
                        a0 = a0 + rows[kk, pl.ds(_off, 16)] * w
                        a1 = a1 + rows[kk, pl.ds(_off + 16, 16)] * w
                    return (a0, a1)

                a0, a1 = lax.fori_loop(
                    0, _NC // 16, inner,
                    (jnp.zeros((16,), jnp.float32),
                     jnp.zeros((16,), jnp.float32)))
                ro = ch * _CH + i
                outb[ro, pl.ds(0, 16)] = a0
                outb[ro, pl.ds(16, 16)] = a1
            return carry

        lax.fori_loop(0, nchunk, chunk_body, 0)
        pltpu.sync_copy(outb, out_hbm.at[pl.ds(base, rpw)])

    return sc_gather


def kernel(query, reference_points, temporal_offsets, input_flatten,
           input_spatial_shapes, input_level_start_index,
           Wv, bv, Wso, bso, Waw, baw, Wtso, btso, Wtaw, btaw, Wo, bo):
    q = query[0]
    ref = reference_points[0]
    toff = temporal_offsets[0]
    xin = input_flatten[0]
    shapes = jnp.asarray(input_spatial_shapes).astype(jnp.int32)
    starts = jnp.asarray(input_level_start_index).astype(jnp.int32)

    len_q = q.shape[0]
    len_in = xin.shape[0]

    wlg = jnp.concatenate([Waw, Wtaw], axis=1)[:, _PERM_LG]
    blg = jnp.concatenate([baw, btaw])[_PERM_LG]
    wxy = jnp.concatenate([Wso, Wtso], axis=1)
    bxy = jnp.concatenate([bso, btso])
    wcat = jnp.concatenate([wlg, wxy[:, _PERM_X], wxy[:, _PERM_Y]], axis=1)
    bcat = jnp.concatenate([blg, bxy[_PERM_X], bxy[_PERM_Y]]).reshape(1, 768)

    wlv = shapes[:, 2][_LVL].astype(jnp.float32)[None, :]
    hlv = shapes[:, 1][_LVL].astype(jnp.float32)[None, :]
    stv = starts[_LVL][None, :]
    headv = jnp.asarray(_HEAD)[None, :]
    dtv = jnp.asarray(_DT)[None, :]
    t_frames = shapes[0, 0]
    nq = len_q // t_frames
    tq = (jnp.arange(len_q, dtype=jnp.int32) // nq)[:, None]
    tm1 = jnp.broadcast_to(t_frames - 1, (1, 256)).astype(jnp.int32)

    rtx = jnp.concatenate(
        [ref[:, :, 0], toff[:, :, :, 0].reshape(len_q, _TW * _NL),
         jnp.zeros((len_q, 4), jnp.float32)], axis=1)
    rty = jnp.concatenate(
        [ref[:, :, 1], toff[:, :, :, 1].reshape(len_q, _TW * _NL),
         jnp.zeros((len_q, 4), jnp.float32)], axis=1)

    value = _matmul(xin, Wv, bv, mb=2176)
    value_rows = value.reshape(len_in * 2, 128)

    idx4, wgt4 = _prep(len_q, q, wcat, bcat, rtx, rty,
                       jnp.asarray(_SEL), jnp.asarray(_BD),
                       wlv, hlv, stv, headv, dtv, tm1, tq)
    idx = idx4.reshape(4, len_q, _NH, _NPTS).transpose(1, 2, 0, 3)
    idx = idx.reshape(len_q * _NH, _NC)
    wgt = wgt4.reshape(4, len_q, _NH, _NPTS).transpose(1, 2, 0, 3)
    wgt = wgt.reshape(len_q * _NH, _NC)

    nrows = len_q * _NH
    nrows_pad = -(-nrows // (_NWORK * _CH)) * (_NWORK * _CH)
    pad = nrows_pad - nrows
    idx = jnp.pad(idx, ((0, pad), (0, 0)))
    wgt = jnp.pad(wgt, ((0, pad), (0, 0)))

    sc_gather = _make_sc_gather(nrows_pad)
    out_rows = sc_gather(value_rows, idx, wgt)[:nrows]

    out_flat = out_rows.reshape(len_q, _DM)
    res = _matmul(out_flat, Wo, bo, mb=len_q)
    return res[None]


# speedup vs baseline: 1.6481x; 1.0995x over previous
import functools

import numpy as np
import jax
import jax.numpy as jnp
from jax import lax
from jax.experimental import pallas as pl
from jax.experimental.pallas import tpu as pltpu
from jax.experimental.pallas import tpu_sc as plsc

_NH, _NL, _NP, _TW, _NTP = 8, 4, 4, 2, 2
_DM = 256
_DH = _DM // _NH
_NPTS = _NL * (_NP + _TW * _NTP)
_NC = 4 * _NPTS


def _build_static():
    lvl = np.zeros(256, np.int32)
    head = np.zeros(256, np.int32)
    dt = np.zeros(256, np.int32)
    perm_lg = np.zeros(256, np.int64)
    perm_x = np.zeros(256, np.int64)
    sel = np.zeros((16, 256), np.float32)
    for c in range(256):
        h, j = divmod(c, 32)
        l, k = divmod(j, 8)
        lvl[c] = l
        head[c] = h
        sel[l, c] = 1.0
        if k < 4:
            p = k
            perm_lg[c] = h * 16 + l * 4 + p
            perm_x[c] = ((h * 4 + l) * 4 + p) * 2
        else:
            w, tp = divmod(k - 4, _NTP)
            dt[c] = -1 if w == 0 else 1
            perm_lg[c] = 128 + h * 16 + w * 8 + l * 2 + tp
            perm_x[c] = 256 + ((((h * 2 + w) * 4 + l) * 2 + tp) * 2)
            sel[4 + w * 4 + l, c] = 1.0
    perm_y = perm_x + 1
    grp = np.arange(256) // 32
    bd = (grp[:, None] == grp[None, :]).astype(np.float32)
    return lvl, head, dt, perm_lg, perm_x, perm_y, sel, bd


_LVL, _HEAD, _DT, _PERM_LG, _PERM_X, _PERM_Y, _SEL, _BD = _build_static()


def _mm_body(x_ref, w_ref, b_ref, o_ref):
    o_ref[...] = jnp.dot(x_ref[...], w_ref[...], precision=lax.Precision.HIGHEST,
                         preferred_element_type=jnp.float32) + b_ref[...]


def _matmul(x, w, b, mb):
    m, kdim = x.shape
    n = w.shape[1]
    return pl.pallas_call(
        _mm_body,
        grid=(m // mb,),
        in_specs=[pl.BlockSpec((mb, kdim), lambda i: (i, 0)),
                  pl.BlockSpec((kdim, n), lambda i: (0, 0)),
                  pl.BlockSpec((1, n), lambda i: (0, 0))],
        out_specs=pl.BlockSpec((mb, n), lambda i: (i, 0)),
        out_shape=jax.ShapeDtypeStruct((m, n), jnp.float32),
    )(x, w, b.reshape(1, n))


def _prep_body(q_ref, wcat_ref, bcat_ref, rtx_ref, rty_ref, sel_ref, bd_ref,
               wl_ref, hl_ref, st_ref, head_ref, dt_ref, tm1_ref, tq_ref,
               idx_ref, wgt_ref):
    p = jnp.dot(q_ref[...], wcat_ref[...], precision=lax.Precision.HIGHEST,
                preferred_element_type=jnp.float32) + bcat_ref[...]
    lg = p[:, :256]
    sx = p[:, 256:512]
    sy = p[:, 512:768]
    m = jnp.max(lg, axis=1, keepdims=True)
    e = jnp.exp(lg - m)
    den = jnp.dot(e, bd_ref[...], precision=lax.Precision.HIGHEST,
                    preferred_element_type=jnp.float32)
    a = e / den
    rx = jnp.dot(rtx_ref[...], sel_ref[...], precision=lax.Precision.HIGHEST,
                 preferred_element_type=jnp.float32)
    ry = jnp.dot(rty_ref[...], sel_ref[...], precision=lax.Precision.HIGHEST,
                 preferred_element_type=jnp.float32)
    wl = wl_ref[...]
    hl = hl_ref[...]
    x = rx * wl + sx - 0.5
    y = ry * hl + sy - 0.5
    x0f = jnp.floor(x)
    y0f = jnp.floor(y)
    x0 = x0f.astype(jnp.int32)
    y0 = y0f.astype(jnp.int32)
    wx1 = x - x0f
    wx0 = 1.0 - wx1
    wy1 = y - y0f
    wy0 = 1.0 - wy1
    wli = wl.astype(jnp.int32)
    hli = hl.astype(jnp.int32)
    t = jnp.clip(tq_ref[...] + dt_ref[...], 0, tm1_ref[...])
    sti = st_ref[...]
    headi = head_ref[...]
    for ci, (dx, dy) in enumerate(((0, 0), (1, 0), (0, 1), (1, 1))):
        xi = x0 + dx
        yi = y0 + dy
        valid = (xi >= 0) & (xi < wli) & (yi >= 0) & (yi < hli)
        xc = jnp.clip(xi, 0, wli - 1)
        yc = jnp.clip(yi, 0, hli - 1)
        flat = jnp.right_shift(
            (sti + (t * hli + yc) * wli + xc) * _NH + headi, 2)
        wx = wx1 if dx else wx0
        wy = wy1 if dy else wy0
        idx_ref[ci] = flat
        wgt_ref[ci] = a * wx * wy * valid.astype(jnp.float32)


def _prep(len_q, q, wcat, bcat, rtx, rty, sel, bd, wlv, hlv, stv, headv, dtv,
          tm1, tq):
    return pl.pallas_call(
        _prep_body,
        out_shape=(jax.ShapeDtypeStruct((4, len_q, 256), jnp.int32),
                   jax.ShapeDtypeStruct((4, len_q, 256), jnp.float32)),
    )(q, wcat, bcat, rtx, rty, sel, bd, wlv, hlv, stv, headv, dtv, tm1, tq)


_NWORK = 32
_CH = 8


_GSZ = 2
_NGRP = _CH // _GSZ


def _make_sc_gather(nrows):
    rpw = nrows // _NWORK
    nchunk = rpw // _CH
    mesh = plsc.VectorSubcoreMesh(core_axis_name="c", subcore_axis_name="s")

    @functools.partial(
        pl.kernel,
        mesh=mesh,
        out_type=jax.ShapeDtypeStruct((nrows, _DH), jnp.float32),
        scratch_types=[
            pltpu.VMEM((2, _CH, _NC), jnp.int32),
            pltpu.VMEM((2, _CH, _NC), jnp.float32),
            pltpu.VMEM((2, _GSZ, _NC, 128), jnp.float32),
            pltpu.VMEM((rpw, _DH), jnp.float32),
            pltpu.SemaphoreType.DMA,
            pltpu.SemaphoreType.DMA,
        ],
    )
    def sc_gather(value_hbm, idx_hbm, wgt_hbm, out_hbm,
                  idxb, wgtb, rows, outb, sem0, sem1):
        cid = lax.axis_index("c")
        sid = lax.axis_index("s")
        wid = sid * 2 + cid
        base = wid * rpw
        sems = (sem0, sem1)

        def fire(cb, g, buf):
            for u in range(_GSZ):
                pltpu.async_copy(
                    value_hbm.at[idxb.at[cb, g * _GSZ + u]],
                    rows.at[buf, u], sems[buf])

        def drain(buf):
            for u in range(_GSZ):
                pltpu.make_async_copy(
                    value_hbm.at[idxb.at[0, u]],
                    rows.at[buf, u], sems[buf]).wait()

        pltpu.sync_copy(idx_hbm.at[pl.ds(base, _CH)], idxb.at[0])
        pltpu.sync_copy(wgt_hbm.at[pl.ds(base, _CH)], wgtb.at[0])
        fire(0, 0, 0)

        def chunk_body(ch, carry):
            cb = lax.rem(ch, 2)
            nb = lax.rem(ch + 1, 2)
            nxt = base + lax.rem(ch + 1, nchunk) * _CH
            pltpu.sync_copy(idx_hbm.at[pl.ds(nxt, _CH)], idxb.at[nb])
            pltpu.sync_copy(wgt_hbm.at[pl.ds(nxt, _CH)], wgtb.at[nb])
            for g in range(_NGRP):
                buf = g % 2
                obuf = (g + 1) % 2
                if g + 1 < _NGRP:
                    fire(cb, g + 1, obuf)
                else:
                    fire(nb, 0, obuf)
                drain(buf)
                for u in range(_GSZ):
                    i = g * _GSZ + u
                    off = (i % 4) * _DH

                    def inner(gg, accs, _u=u, _off=off, _buf=buf, _i=i):
                        a0, a1 = accs
                        w16 = wgtb[cb, _i, pl.ds(gg * 16, 16)]
                        for lane in range(16):
                            w = w16[lane]
                            kk = gg * 16 + lane
                            a0 = a0 + rows[_buf, _u, kk,
                                           pl.ds(_off, 16)] * w
                            a1 = a1 + rows[_buf, _u, kk,
                                           pl.ds(_off + 16, 16)] * w
                        return (a0, a1)

                    a0, a1 = lax.fori_loop(
                        0, _NC // 16, inner,
                        (jnp.zeros((16,), jnp.float32),
                         jnp.zeros((16,), jnp.float32)))
                    ro = ch * _CH + i
                    outb[ro, pl.ds(0, 16)] = a0
                    outb[ro, pl.ds(16, 16)] = a1
            return carry

        lax.fori_loop(0, nchunk, chunk_body, 0)
        drain(0)

        pltpu.sync_copy(outb, out_hbm.at[pl.ds(base, rpw)])

    return sc_gather


def kernel(query, reference_points, temporal_offsets, input_flatten,
           input_spatial_shapes, input_level_start_index,
           Wv, bv, Wso, bso, Waw, baw, Wtso, btso, Wtaw, btaw, Wo, bo):
    q = query[0]
    ref = reference_points[0]
    toff = temporal_offsets[0]
    xin = input_flatten[0]
    shapes = jnp.asarray(input_spatial_shapes).astype(jnp.int32)
    starts = jnp.asarray(input_level_start_index).astype(jnp.int32)

    len_q = q.shape[0]
    len_in = xin.shape[0]

    wlg = jnp.concatenate([Waw, Wtaw], axis=1)[:, _PERM_LG]
    blg = jnp.concatenate([baw, btaw])[_PERM_LG]
    wxy = jnp.concatenate([Wso, Wtso], axis=1)
    bxy = jnp.concatenate([bso, btso])
    wcat = jnp.concatenate([wlg, wxy[:, _PERM_X], wxy[:, _PERM_Y]], axis=1)
    bcat = jnp.concatenate([blg, bxy[_PERM_X], bxy[_PERM_Y]]).reshape(1, 768)

    wlv = shapes[:, 2][_LVL].astype(jnp.float32)[None, :]
    hlv = shapes[:, 1][_LVL].astype(jnp.float32)[None, :]
    stv = starts[_LVL][None, :]
    headv = jnp.asarray(_HEAD)[None, :]
    dtv = jnp.asarray(_DT)[None, :]
    t_frames = shapes[0, 0]
    nq = len_q // t_frames
    tq = (jnp.arange(len_q, dtype=jnp.int32) // nq)[:, None]
    tm1 = jnp.broadcast_to(t_frames - 1, (1, 256)).astype(jnp.int32)

    rtx = jnp.concatenate(
        [ref[:, :, 0], toff[:, :, :, 0].reshape(len_q, _TW * _NL),
         jnp.zeros((len_q, 4), jnp.float32)], axis=1)
    rty = jnp.concatenate(
        [ref[:, :, 1], toff[:, :, :, 1].reshape(len_q, _TW * _NL),
         jnp.zeros((len_q, 4), jnp.float32)], axis=1)

    value = _matmul(xin, Wv, bv, mb=2176)
    value_rows = value.reshape(len_in * 2, 128)

    idx4, wgt4 = _prep(len_q, q, wcat, bcat, rtx, rty,
                       jnp.asarray(_SEL), jnp.asarray(_BD),
                       wlv, hlv, stv, headv, dtv, tm1, tq)
    idx = idx4.reshape(4, len_q, _NH, _NPTS).transpose(1, 2, 0, 3)
    idx = idx.reshape(len_q * _NH, _NC)
    wgt = wgt4.reshape(4, len_q, _NH, _NPTS).transpose(1, 2, 0, 3)
    wgt = wgt.reshape(len_q * _NH, _NC)

    nrows = len_q * _NH
    nrows_pad = -(-nrows // (_NWORK * _CH)) * (_NWORK * _CH)
    pad = nrows_pad - nrows
    idx = jnp.pad(idx, ((0, pad), (0, 0)))
    wgt = jnp.pad(wgt, ((0, pad), (0, 0)))

    sc_gather = _make_sc_gather(nrows_pad)
    out_rows = sc_gather(value_rows, idx, wgt)[:nrows]

    out_flat = out_rows.reshape(len_q, _DM)
    res = _matmul(out_flat, Wo, bo, mb=len_q)
    return res[None]
